# Initial kernel scaffold; baseline (speedup 1.0000x reference)
import math

import jax
import jax.numpy as jnp
from jax import lax
from jax.experimental import pallas as pl
from jax.experimental.pallas import tpu as pltpu
from jax.experimental.pallas import tpu_sc as plsc

_HALF_PI = math.pi / 2
_NC, _NS, _LANES = 2, 16, 16
_NW = _NC * _NS
_NBUF = 4


def _pick_chunk(per_w: int) -> int:
    for cand in range(128, 0, -8):
        if per_w % cand == 0:
            return cand
    raise ValueError(f"no valid chunk size for per-worker count {per_w}")


def _make_lookup(num_rows: int, dim: int):
    assert num_rows % _NW == 0
    per_w = num_rows // _NW
    chunk = _pick_chunk(per_w)
    nchunk = per_w // chunk
    nbuf = _NBUF if nchunk % _NBUF == 0 and nchunk > _NBUF else 1
    vecs_per_row = dim // _LANES
    assert dim % _LANES == 0

    mesh = plsc.VectorSubcoreMesh(core_axis_name="c", subcore_axis_name="s")

    def body(x_hbm, w_hbm, out_hbm, idx_v, *bufs_and_sems):
        rows = bufs_and_sems[:nbuf]
        gsems = bufs_and_sems[nbuf:2 * nbuf]
        ssems = bufs_and_sems[2 * nbuf:3 * nbuf]

        wid = lax.axis_index("s") * _NC + lax.axis_index("c")
        base = wid * per_w
        pltpu.sync_copy(x_hbm.at[pl.ds(base, per_w)], idx_v)

        def gather(c, b):
            return pltpu.make_async_copy(
                w_hbm.at[idx_v.at[pl.ds(c * chunk, chunk)]], rows[b], gsems[b])

        def store(c, b):
            return pltpu.make_async_copy(
                rows[b], out_hbm.at[pl.ds(base + c * chunk, chunk)], ssems[b])

        def scale(b):
            buf = rows[b]

            def row_fn(i, carry):
                for j in range(vecs_per_row):
                    sl = pl.ds(j * _LANES, _LANES)
                    buf[i, sl] = buf[i, sl] * _HALF_PI
                return carry

            lax.fori_loop(0, chunk, row_fn, 0)

        def step(c, b):
            gather(c, b).wait()
            scale(b)
            store(c, b).start()

        for b in range(nbuf):
            gather(b, b).start()

        if nchunk > nbuf:
            def outer(g, carry):
                for b in range(nbuf):
                    c = g * nbuf + b
                    step(c, b)
                    store(c, b).wait()
                    gather(c + nbuf, b).start()
                return carry

            lax.fori_loop(0, nchunk // nbuf - 1, outer, 0)

        for b in range(nbuf):
            c = nchunk - nbuf + b
            step(c, b)
        for b in range(nbuf):
            store(nchunk - nbuf + b, b).wait()

    scratch = [pltpu.VMEM((per_w,), jnp.int32)]
    scratch += [pltpu.VMEM((chunk, dim), jnp.float32) for _ in range(nbuf)]
    scratch += [pltpu.SemaphoreType.DMA for _ in range(2 * nbuf)]

    return pl.kernel(
        body,
        out_type=jax.ShapeDtypeStruct((num_rows, dim), jnp.float32),
        mesh=mesh,
        scratch_types=scratch,
    )


def kernel(x, weight):
    b0, b1 = x.shape
    num_rows = b0 * b1
    dim = weight.shape[1]
    xf = x.reshape(num_rows).astype(jnp.int32)
    out = _make_lookup(num_rows, dim)(xf, weight)
    return out.reshape(b0, b1, dim)


# speedup vs baseline: 1.2568x; 1.2568x over previous
import math

import jax
import jax.numpy as jnp
from jax import lax
from jax.experimental import pallas as pl
from jax.experimental.pallas import tpu as pltpu
from jax.experimental.pallas import tpu_sc as plsc

_HALF_PI = math.pi / 2
_NC, _NS, _LANES = 2, 16, 16
_NW = _NC * _NS
_NBUF = 4


def _pick_chunk(per_w: int) -> int:
    for want_ring in (True, False):
        for cand in range(128, 0, -8):
            if per_w % cand == 0 and (
                    not want_ring or (per_w // cand) % _NBUF == 0):
                return cand
    raise ValueError(f"no valid chunk size for per-worker count {per_w}")


def _make_lookup(num_rows: int, dim: int):
    assert num_rows % _NW == 0
    per_w = num_rows // _NW
    chunk = _pick_chunk(per_w)
    nchunk = per_w // chunk
    nbuf = _NBUF if nchunk % _NBUF == 0 and nchunk > _NBUF else 1
    vecs_per_row = dim // _LANES
    assert dim % _LANES == 0

    mesh = plsc.VectorSubcoreMesh(core_axis_name="c", subcore_axis_name="s")

    def body(x_hbm, w_hbm, out_hbm, idx_v, *bufs_and_sems):
        rows = bufs_and_sems[:nbuf]
        gsems = bufs_and_sems[nbuf:2 * nbuf]
        ssems = bufs_and_sems[2 * nbuf:3 * nbuf]

        wid = lax.axis_index("s") * _NC + lax.axis_index("c")
        base = wid * per_w
        pltpu.sync_copy(x_hbm.at[pl.ds(base, per_w)], idx_v)

        def gather(c, b):
            return pltpu.make_async_copy(
                w_hbm.at[idx_v.at[pl.ds(c * chunk, chunk)]], rows[b], gsems[b])

        def store(c, b):
            return pltpu.make_async_copy(
                rows[b], out_hbm.at[pl.ds(base + c * chunk, chunk)], ssems[b])

        def scale(b):
            buf = rows[b]

            def row_fn(i, carry):
                for j in range(vecs_per_row):
                    sl = pl.ds(j * _LANES, _LANES)
                    buf[i, sl] = buf[i, sl] * _HALF_PI
                return carry

            lax.fori_loop(0, chunk, row_fn, 0)

        def step(c, b):
            gather(c, b).wait()
            scale(b)
            store(c, b).start()

        for b in range(nbuf):
            gather(b, b).start()

        if nchunk > nbuf:
            def outer(g, carry):
                for b in range(nbuf):
                    c = g * nbuf + b
                    step(c, b)
                    store(c, b).wait()
                    gather(c + nbuf, b).start()
                return carry

            lax.fori_loop(0, nchunk // nbuf - 1, outer, 0)

        for b in range(nbuf):
            c = nchunk - nbuf + b
            step(c, b)
        for b in range(nbuf):
            store(nchunk - nbuf + b, b).wait()

    scratch = [pltpu.VMEM((per_w,), jnp.int32)]
    scratch += [pltpu.VMEM((chunk, dim), jnp.float32) for _ in range(nbuf)]
    scratch += [pltpu.SemaphoreType.DMA for _ in range(2 * nbuf)]

    return pl.kernel(
        body,
        out_type=jax.ShapeDtypeStruct((num_rows, dim), jnp.float32),
        mesh=mesh,
        scratch_types=scratch,
        compiler_params=pltpu.CompilerParams(use_tc_tiling_on_sc=False),
    )


def kernel(x, weight):
    b0, b1 = x.shape
    num_rows = b0 * b1
    dim = weight.shape[1]
    xf = x.reshape(num_rows).astype(jnp.int32)
    out = _make_lookup(num_rows, dim)(xf, weight)
    return out.reshape(b0, b1, dim)


# speedup vs baseline: 1.3016x; 1.0357x over previous
import math

import jax
import jax.numpy as jnp
from jax import lax
from jax.experimental import pallas as pl
from jax.experimental.pallas import tpu as pltpu
from jax.experimental.pallas import tpu_sc as plsc

_HALF_PI = math.pi / 2
_NC, _NS, _LANES = 2, 16, 16
_NW = _NC * _NS
_NBUF = 8


def _pick_chunk(per_w: int) -> int:
    for want_ring in (True, False):
        for cand in range(128, 0, -8):
            if per_w % cand == 0 and (
                    not want_ring or (per_w // cand) % _NBUF == 0):
                return cand
    raise ValueError(f"no valid chunk size for per-worker count {per_w}")


def _make_lookup(num_rows: int, dim: int):
    assert num_rows % _NW == 0
    per_w = num_rows // _NW
    chunk = _pick_chunk(per_w)
    nchunk = per_w // chunk
    nbuf = _NBUF if nchunk % _NBUF == 0 and nchunk > _NBUF else 1
    vecs_per_row = dim // _LANES
    assert dim % _LANES == 0

    mesh = plsc.VectorSubcoreMesh(core_axis_name="c", subcore_axis_name="s")

    def body(x_hbm, w_hbm, out_hbm, idx_v, *bufs_and_sems):
        rows = bufs_and_sems[:nbuf]
        gsems = bufs_and_sems[nbuf:2 * nbuf]
        ssems = bufs_and_sems[2 * nbuf:3 * nbuf]

        wid = lax.axis_index("s") * _NC + lax.axis_index("c")
        base = wid * per_w
        pltpu.sync_copy(x_hbm.at[pl.ds(base, per_w)], idx_v)

        def gather(c, b):
            return pltpu.make_async_copy(
                w_hbm.at[idx_v.at[pl.ds(c * chunk, chunk)]], rows[b], gsems[b])

        def store(c, b):
            return pltpu.make_async_copy(
                rows[b], out_hbm.at[pl.ds(base + c * chunk, chunk)], ssems[b])

        def scale(b):
            buf = rows[b]

            @plsc.parallel_loop(0, chunk, unroll=4)
            def _(i):
                for j in range(vecs_per_row):
                    sl = pl.ds(j * _LANES, _LANES)
                    buf[i, sl] = buf[i, sl] * _HALF_PI

        def step(c, b):
            gather(c, b).wait()
            scale(b)
            store(c, b).start()

        for b in range(nbuf):
            gather(b, b).start()

        if nchunk > nbuf:
            def outer(g, carry):
                for b in range(nbuf):
                    c = g * nbuf + b
                    step(c, b)
                    store(c, b).wait()
                    gather(c + nbuf, b).start()
                return carry

            lax.fori_loop(0, nchunk // nbuf - 1, outer, 0)

        for b in range(nbuf):
            c = nchunk - nbuf + b
            step(c, b)
        for b in range(nbuf):
            store(nchunk - nbuf + b, b).wait()

    scratch = [pltpu.VMEM((per_w,), jnp.int32)]
    scratch += [pltpu.VMEM((chunk, dim), jnp.float32) for _ in range(nbuf)]
    scratch += [pltpu.SemaphoreType.DMA for _ in range(2 * nbuf)]

    return pl.kernel(
        body,
        out_type=jax.ShapeDtypeStruct((num_rows, dim), jnp.float32),
        mesh=mesh,
        scratch_types=scratch,
        compiler_params=pltpu.CompilerParams(use_tc_tiling_on_sc=False),
    )


def kernel(x, weight):
    b0, b1 = x.shape
    num_rows = b0 * b1
    dim = weight.shape[1]
    xf = x.reshape(num_rows).astype(jnp.int32)
    out = _make_lookup(num_rows, dim)(xf, weight)
    return out.reshape(b0, b1, dim)
